# baseline (speedup 1.0000x reference)
import jax
import jax.numpy as jnp
from jax.experimental import pallas as pl
from jax.experimental.pallas import tpu as pltpu

N = 10000
E = 160000
D_IN = 128
D_E = 16
D_EMB = 64
D = 32
L = 3
B = 16
NB = 625


def _leaky(x):
    return jnp.where(x > 0, x, 0.01 * x)


def _nnconv(h, src, dst, eattr, Wedge, bedge, Wroot, broot):
    hs = h[src]
    acc = hs @ bedge
    for k in range(D_E):
        acc = acc + eattr[:, k:k + 1] * (hs @ Wedge[k])
    seg = jax.ops.segment_sum(acc, dst, num_segments=N)
    deg = jax.ops.segment_sum(jnp.ones((E, 1), h.dtype), dst, num_segments=N)
    mean = seg / jnp.maximum(deg, 1.0)
    return mean + h @ Wroot + broot


def _encode(x, ei, ea, Wn, bn, We, be, Wh, bh, Wedge, bedge, Wroot, broot, Wout, bout):
    node = x @ Wn + bn
    e = ea @ We + be
    hid = _leaky(node) @ Wh + bh
    src, dst = ei[0], ei[1]
    prev = []
    for l in range(L):
        h_in = _leaky(hid)
        conv = _nnconv(h_in, src, dst, e, Wedge[l], bedge[l], Wroot[l], broot[l])
        hid = _leaky(conv) @ Wout[l] + bout[l]
        if l - 2 >= 0:
            hid = hid + prev[l - 2]
        prev.append(hid)
    return hid


def _outer_body(lig_ref, rec_ref, ops_ref, out_ref):
    lig = lig_ref[0]
    rec = rec_ref[0]
    ops = jax.lax.dot_general(lig, rec, (((1,), (1,)), ((), ())),
                              preferred_element_type=jnp.float32)
    ops_ref[0] = ops
    out_ref[0, 0, 0] = jnp.sum(ops) / (NB * NB)


def _outer(lig_hid, rec_hid):
    lig3 = lig_hid.reshape(B, NB, D)
    rec3 = rec_hid.reshape(B, NB, D)
    ops, out = pl.pallas_call(
        _outer_body,
        grid=(B,),
        in_specs=[
            pl.BlockSpec((1, NB, D), lambda b: (b, 0, 0)),
            pl.BlockSpec((1, NB, D), lambda b: (b, 0, 0)),
        ],
        out_specs=[
            pl.BlockSpec((1, NB, NB), lambda b: (b, 0, 0)),
            pl.BlockSpec((1, 1, 1), lambda b: (b, 0, 0), memory_space=pltpu.SMEM),
        ],
        out_shape=[
            jax.ShapeDtypeStruct((B, NB, NB), jnp.float32),
            jax.ShapeDtypeStruct((B, 1, 1), jnp.float32),
        ],
    )(lig3, rec3)
    return out.reshape(B), ops


def kernel(lig_x, lig_edge_index, lig_edge_attr, rec_x, rec_edge_index, rec_edge_attr,
           lig_emb_Wn, lig_emb_bn, lig_emb_We, lig_emb_be, lig_hid_W, lig_hid_b,
           lig_Wedge, lig_bedge, lig_Wroot, lig_broot, lig_Wout, lig_bout,
           rec_emb_Wn, rec_emb_bn, rec_emb_We, rec_emb_be, rec_hid_W, rec_hid_b,
           rec_Wedge, rec_bedge, rec_Wroot, rec_broot, rec_Wout, rec_bout):
    lig_hid = _encode(lig_x, lig_edge_index, lig_edge_attr, lig_emb_Wn, lig_emb_bn,
                      lig_emb_We, lig_emb_be, lig_hid_W, lig_hid_b, lig_Wedge,
                      lig_bedge, lig_Wroot, lig_broot, lig_Wout, lig_bout)
    rec_hid = _encode(rec_x, rec_edge_index, rec_edge_attr, rec_emb_Wn, rec_emb_bn,
                      rec_emb_We, rec_emb_be, rec_hid_W, rec_hid_b, rec_Wedge,
                      rec_bedge, rec_Wroot, rec_broot, rec_Wout, rec_bout)
    return _outer(lig_hid, rec_hid)


# speedup vs baseline: 1.4202x; 1.4202x over previous
import functools

import jax
import jax.numpy as jnp
from jax import lax
from jax.experimental import pallas as pl
from jax.experimental.pallas import tpu as pltpu
from jax.experimental.pallas import tpu_sc as plsc

N = 10000
E = 160000
D_IN = 128
D_E = 16
D_EMB = 64
D = 32
L = 3
B = 16
NB = 625

NSC = 2
NTEC = 16
EPT = E // NTEC
CH = 80
NCH = EPT // CH
SUP = 25
NSUP = NCH // SUP
ZR = 125
NPT = N // NTEC


def _leaky(x):
    return jnp.where(x > 0, x, 0.01 * x)


_MESH = plsc.VectorSubcoreMesh(core_axis_name="c", subcore_axis_name="s")
_SC_PARAMS = pltpu.CompilerParams(use_tc_tiling_on_sc=False)


@functools.partial(
    pl.kernel,
    mesh=_MESH,
    out_type=jax.ShapeDtypeStruct((2 * E, D), jnp.float32),
    scratch_types=[
        pltpu.VMEM((NCH, CH), jnp.int32),
        pltpu.VMEM((SUP * CH, D), jnp.float32),
        pltpu.SemaphoreType.DMA,
    ],
    compiler_params=_SC_PARAMS,
)
def _sc_gather(table, src4, out, idxbuf, vbuf, sem):
    c = lax.axis_index("c")
    s = lax.axis_index("s")
    pltpu.sync_copy(src4.at[c, s], idxbuf)
    off = jnp.full((16,), N, jnp.int32) * c

    def adj(i, carry):
        for q in range(CH // 16):
            idxbuf[i, pl.ds(16 * q, 16)] = idxbuf[i, pl.ds(16 * q, 16)] + off
        return carry

    lax.fori_loop(0, NCH, adj, 0)
    base = (c * NTEC + s) * EPT
    for sup in range(NSUP):
        cps = []
        for m in range(SUP):
            cp = pltpu.async_copy(
                table.at[idxbuf.at[sup * SUP + m]],
                vbuf.at[pl.ds(CH * m, CH)],
                sem,
            )
            cps.append(cp)
        for cp in cps:
            cp.wait()
        pltpu.sync_copy(vbuf, out.at[pl.ds(base + SUP * CH * sup, SUP * CH)])


def _scatter_body(acc5, dst4, outs, idxbuf, vbuf, zbuf, onesbuf, accs, with_deg):
    c = lax.axis_index("c")
    s = lax.axis_index("s")
    pltpu.sync_copy(dst4.at[c, s], idxbuf)
    zero16 = jnp.zeros((16,), jnp.float32)

    def zfill(i, carry):
        for q in range(D // 16):
            zbuf[i, pl.ds(16 * q, 16)] = zero16
        return carry

    lax.fori_loop(0, ZR, zfill, 0)
    if with_deg:
        one16 = jnp.ones((16,), jnp.float32)

        def ofill(i, carry):
            for q in range(D // 16):
                onesbuf[i, pl.ds(16 * q, 16)] = one16
            return carry

        lax.fori_loop(0, CH, ofill, 0)
    for q in range(NPT // ZR):
        for a in accs:
            pltpu.sync_copy(zbuf, a.at[pl.ds(s * NPT + ZR * q, ZR)])
    plsc.subcore_barrier()
    for sup in range(NSUP):
        pltpu.sync_copy(acc5.at[c, s, pl.ds(SUP * sup, SUP)], vbuf)
        for m in range(SUP):
            idxrow = idxbuf.at[sup * SUP + m]
            pltpu.sync_copy(vbuf.at[m], accs[0].at[idxrow], add=True)
            if with_deg:
                pltpu.sync_copy(onesbuf, accs[1].at[idxrow], add=True)
    plsc.subcore_barrier()
    for a, o in zip(accs, outs):
        pltpu.sync_copy(a.at[pl.ds(s * NPT, NPT)], o.at[c, pl.ds(s * NPT, NPT)])


@functools.partial(
    pl.kernel,
    mesh=_MESH,
    out_type=jax.ShapeDtypeStruct((2, N, D), jnp.float32),
    scratch_types=[
        pltpu.VMEM((NCH, CH), jnp.int32),
        pltpu.VMEM((SUP, CH, D), jnp.float32),
        pltpu.VMEM((ZR, D), jnp.float32),
        pltpu.VMEM_SHARED((N, D), jnp.float32),
    ],
    compiler_params=_SC_PARAMS,
)
def _sc_scatter(acc5, dst4, out, idxbuf, vbuf, zbuf, acc_sh):
    _scatter_body(acc5, dst4, [out], idxbuf, vbuf, zbuf, None, [acc_sh], False)


@functools.partial(
    pl.kernel,
    mesh=_MESH,
    out_type=[
        jax.ShapeDtypeStruct((2, N, D), jnp.float32),
        jax.ShapeDtypeStruct((2, N, D), jnp.float32),
    ],
    scratch_types=[
        pltpu.VMEM((NCH, CH), jnp.int32),
        pltpu.VMEM((SUP, CH, D), jnp.float32),
        pltpu.VMEM((ZR, D), jnp.float32),
        pltpu.VMEM((CH, D), jnp.float32),
        pltpu.VMEM_SHARED((N, D), jnp.float32),
        pltpu.VMEM_SHARED((N, D), jnp.float32),
    ],
    compiler_params=_SC_PARAMS,
)
def _sc_scatter_deg(acc5, dst4, seg_out, deg_out, idxbuf, vbuf, zbuf, onesbuf,
                    acc_sh, deg_sh):
    _scatter_body(acc5, dst4, [seg_out, deg_out], idxbuf, vbuf, zbuf, onesbuf,
                  [acc_sh, deg_sh], True)


def _pre_body(x_ref, Wn_ref, bn_ref, Wh_ref, bh_ref, hin_ref):
    node = jax.lax.dot_general(x_ref[0], Wn_ref[0], (((1,), (0,)), ((), ())),
                               preferred_element_type=jnp.float32) + bn_ref[0]
    hid = jax.lax.dot_general(_leaky(node), Wh_ref[0], (((1,), (0,)), ((), ())),
                              preferred_element_type=jnp.float32) + bh_ref[0]
    hin_ref[0] = _leaky(hid)


def _tc_pre(xs, Wn, bn, Wh, bh):
    BN = 2000
    return pl.pallas_call(
        _pre_body,
        grid=(2, N // BN),
        in_specs=[
            pl.BlockSpec((1, BN, D_IN), lambda g, i: (g, i, 0)),
            pl.BlockSpec((1, D_IN, D_EMB), lambda g, i: (g, 0, 0)),
            pl.BlockSpec((1, 1, D_EMB), lambda g, i: (g, 0, 0)),
            pl.BlockSpec((1, D_EMB, D), lambda g, i: (g, 0, 0)),
            pl.BlockSpec((1, 1, D), lambda g, i: (g, 0, 0)),
        ],
        out_specs=pl.BlockSpec((1, BN, D), lambda g, i: (g, i, 0)),
        out_shape=jax.ShapeDtypeStruct((2, N, D), jnp.float32),
    )(xs, Wn, bn, Wh, bh)


def _edge_body(ea_ref, hs_ref, Wall_ref, We_ref, be_ref, out_ref):
    e = jax.lax.dot_general(ea_ref[0], We_ref[0], (((1,), (0,)), ((), ())),
                            preferred_element_type=jnp.float32) + be_ref[0]
    P = jax.lax.dot_general(hs_ref[0], Wall_ref[0], (((1,), (0,)), ((), ())),
                            preferred_element_type=jnp.float32)
    acc = P[:, :D]
    for k in range(D_E):
        acc = acc + e[:, k:k + 1] * P[:, D * (k + 1):D * (k + 2)]
    out_ref[0] = acc


def _tc_edge(eas, hs3, Wall_l, We, be):
    BE = 2000
    return pl.pallas_call(
        _edge_body,
        grid=(2, E // BE),
        in_specs=[
            pl.BlockSpec((1, BE, D_E), lambda g, i: (g, i, 0)),
            pl.BlockSpec((1, BE, D), lambda g, i: (g, i, 0)),
            pl.BlockSpec((1, D, 17 * D), lambda g, i: (g, 0, 0)),
            pl.BlockSpec((1, D_E, D_E), lambda g, i: (g, 0, 0)),
            pl.BlockSpec((1, 1, D_E), lambda g, i: (g, 0, 0)),
        ],
        out_specs=pl.BlockSpec((1, BE, D), lambda g, i: (g, i, 0)),
        out_shape=jax.ShapeDtypeStruct((2, E, D), jnp.float32),
    )(eas, hs3, Wall_l, We, be)


def _node_body_plain(seg_ref, deg_ref, hin_ref, Wr_ref, br_ref, Wo_ref, bo_ref,
                     hid_ref, hnext_ref):
    mean = seg_ref[0] / jnp.maximum(deg_ref[0], 1.0)
    conv = mean + jax.lax.dot_general(
        hin_ref[0], Wr_ref[0], (((1,), (0,)), ((), ())),
        preferred_element_type=jnp.float32) + br_ref[0]
    hid = jax.lax.dot_general(_leaky(conv), Wo_ref[0], (((1,), (0,)), ((), ())),
                              preferred_element_type=jnp.float32) + bo_ref[0]
    hid_ref[0] = hid
    hnext_ref[0] = _leaky(hid)


def _node_body_resid(seg_ref, deg_ref, hin_ref, Wr_ref, br_ref, Wo_ref, bo_ref,
                     res_ref, hid_ref, hnext_ref):
    mean = seg_ref[0] / jnp.maximum(deg_ref[0], 1.0)
    conv = mean + jax.lax.dot_general(
        hin_ref[0], Wr_ref[0], (((1,), (0,)), ((), ())),
        preferred_element_type=jnp.float32) + br_ref[0]
    hid = jax.lax.dot_general(_leaky(conv), Wo_ref[0], (((1,), (0,)), ((), ())),
                              preferred_element_type=jnp.float32) + bo_ref[0]
    hid = hid + res_ref[0]
    hid_ref[0] = hid
    hnext_ref[0] = _leaky(hid)


def _tc_node(seg, deg, hin, Wr, br, Wo, bo, resid=None):
    BN = 2000
    nmap = lambda g, i: (g, i, 0)
    wmap = lambda g, i: (g, 0, 0)
    in_specs = [
        pl.BlockSpec((1, BN, D), nmap),
        pl.BlockSpec((1, BN, D), nmap),
        pl.BlockSpec((1, BN, D), nmap),
        pl.BlockSpec((1, D, D), wmap),
        pl.BlockSpec((1, 1, D), wmap),
        pl.BlockSpec((1, D, D), wmap),
        pl.BlockSpec((1, 1, D), wmap),
    ]
    args = [seg, deg, hin, Wr, br, Wo, bo]
    body = _node_body_plain
    if resid is not None:
        in_specs.append(pl.BlockSpec((1, BN, D), nmap))
        args.append(resid)
        body = _node_body_resid
    return pl.pallas_call(
        body,
        grid=(2, N // BN),
        in_specs=in_specs,
        out_specs=[
            pl.BlockSpec((1, BN, D), nmap),
            pl.BlockSpec((1, BN, D), nmap),
        ],
        out_shape=[
            jax.ShapeDtypeStruct((2, N, D), jnp.float32),
            jax.ShapeDtypeStruct((2, N, D), jnp.float32),
        ],
    )(*args)


def _outer_body(lig_ref, rec_ref, ops_ref, out_ref):
    ops = jax.lax.dot_general(lig_ref[0], rec_ref[0], (((1,), (1,)), ((), ())),
                              preferred_element_type=jnp.float32)
    ops_ref[0] = ops
    out_ref[0, 0, 0] = jnp.sum(ops) / (NB * NB)


def _outer(lig_hid, rec_hid):
    lig3 = lig_hid.reshape(B, NB, D)
    rec3 = rec_hid.reshape(B, NB, D)
    ops, out = pl.pallas_call(
        _outer_body,
        grid=(B,),
        in_specs=[
            pl.BlockSpec((1, NB, D), lambda b: (b, 0, 0)),
            pl.BlockSpec((1, NB, D), lambda b: (b, 0, 0)),
        ],
        out_specs=[
            pl.BlockSpec((1, NB, NB), lambda b: (b, 0, 0)),
            pl.BlockSpec((1, 1, 1), lambda b: (b, 0, 0), memory_space=pltpu.SMEM),
        ],
        out_shape=[
            jax.ShapeDtypeStruct((B, NB, NB), jnp.float32),
            jax.ShapeDtypeStruct((B, 1, 1), jnp.float32),
        ],
    )(lig3, rec3)
    return out.reshape(B), ops


def kernel(lig_x, lig_edge_index, lig_edge_attr, rec_x, rec_edge_index, rec_edge_attr,
           lig_emb_Wn, lig_emb_bn, lig_emb_We, lig_emb_be, lig_hid_W, lig_hid_b,
           lig_Wedge, lig_bedge, lig_Wroot, lig_broot, lig_Wout, lig_bout,
           rec_emb_Wn, rec_emb_bn, rec_emb_We, rec_emb_be, rec_hid_W, rec_hid_b,
           rec_Wedge, rec_bedge, rec_Wroot, rec_broot, rec_Wout, rec_bout):
    f32 = jnp.float32
    xs = jnp.stack([lig_x, rec_x])
    eas = jnp.stack([lig_edge_attr, rec_edge_attr])
    ei = jnp.stack([lig_edge_index, rec_edge_index])
    src4 = ei[:, 0].reshape(2, NTEC, NCH, CH)
    dst4 = ei[:, 1].reshape(2, NTEC, NCH, CH)

    Wn = jnp.stack([lig_emb_Wn, rec_emb_Wn])
    bn = jnp.stack([lig_emb_bn, rec_emb_bn]).reshape(2, 1, D_EMB)
    We = jnp.stack([lig_emb_We, rec_emb_We])
    be = jnp.stack([lig_emb_be, rec_emb_be]).reshape(2, 1, D_E)
    Wh = jnp.stack([lig_hid_W, rec_hid_W])
    bh = jnp.stack([lig_hid_b, rec_hid_b]).reshape(2, 1, D)
    Wedge = jnp.stack([lig_Wedge, rec_Wedge])
    bedge = jnp.stack([lig_bedge, rec_bedge])
    T = jnp.concatenate([bedge[:, :, None], Wedge], axis=2)
    Wall = jnp.transpose(T, (0, 1, 3, 2, 4)).reshape(2, L, D, 17 * D)
    Wroot = jnp.stack([lig_Wroot, rec_Wroot])
    broot = jnp.stack([lig_broot, rec_broot]).reshape(2, L, 1, D)
    Wout = jnp.stack([lig_Wout, rec_Wout])
    bout = jnp.stack([lig_bout, rec_bout]).reshape(2, L, 1, D)

    hin = _tc_pre(xs, Wn, bn, Wh, bh)
    deg = None
    prev0 = None
    hid = None
    for l in range(L):
        table = hin.reshape(2 * N, D)
        hs = _sc_gather(table, src4)
        acc = _tc_edge(eas, hs.reshape(2, E, D), Wall[:, l], We, be)
        acc5 = acc.reshape(2, NTEC, NCH, CH, D)
        if l == 0:
            seg, deg = _sc_scatter_deg(acc5, dst4)
        else:
            seg = _sc_scatter(acc5, dst4)
        resid = prev0 if l == 2 else None
        hid, hin = _tc_node(seg, deg, hin, Wroot[:, l], broot[:, l],
                            Wout[:, l], bout[:, l], resid)
        if l == 0:
            prev0 = hid
    return _outer(hid[0], hid[1])


# speedup vs baseline: 3.5161x; 2.4759x over previous
import functools

import jax
import jax.numpy as jnp
from jax import lax
from jax.experimental import pallas as pl
from jax.experimental.pallas import tpu as pltpu
from jax.experimental.pallas import tpu_sc as plsc

N = 10000
E = 160000
D_IN = 128
D_E = 16
D_EMB = 64
D = 32
L = 3
B = 16
NB = 625

NSC = 2
NTEC = 16
EPT = E // NTEC
CH = 80
NCH = EPT // CH
SUP = 25
NSUP = NCH // SUP
ZR = 125
NPT = N // NTEC


def _leaky(x):
    return jnp.where(x > 0, x, 0.01 * x)


_MESH = plsc.VectorSubcoreMesh(core_axis_name="c", subcore_axis_name="s")
_SC_PARAMS = pltpu.CompilerParams(use_tc_tiling_on_sc=False)


@functools.partial(
    pl.kernel,
    mesh=_MESH,
    out_type=jax.ShapeDtypeStruct((2 * E, D), jnp.float32),
    scratch_types=[
        pltpu.VMEM((NCH, CH), jnp.int32),
        pltpu.VMEM((SUP * CH, D), jnp.float32),
        pltpu.SemaphoreType.DMA,
    ],
    compiler_params=_SC_PARAMS,
)
def _sc_gather(table, src4, out, idxbuf, vbuf, sem):
    c = lax.axis_index("c")
    s = lax.axis_index("s")
    pltpu.sync_copy(src4.at[c, s], idxbuf)
    off = jnp.full((16,), N, jnp.int32) * c

    def adj(i, carry):
        for q in range(CH // 16):
            idxbuf[i, pl.ds(16 * q, 16)] = idxbuf[i, pl.ds(16 * q, 16)] + off
        return carry

    lax.fori_loop(0, NCH, adj, 0)
    base = (c * NTEC + s) * EPT
    for sup in range(NSUP):
        cps = []
        for m in range(SUP):
            cp = pltpu.async_copy(
                table.at[idxbuf.at[sup * SUP + m]],
                vbuf.at[pl.ds(CH * m, CH)],
                sem,
            )
            cps.append(cp)
        for cp in cps:
            cp.wait()
        pltpu.sync_copy(vbuf, out.at[pl.ds(base + SUP * CH * sup, SUP * CH)])


def _scatter_body(acc5, dst4, outs, idxbuf, vbuf, zbuf, onesbuf, accs, with_deg):
    c = lax.axis_index("c")
    s = lax.axis_index("s")
    pltpu.sync_copy(dst4.at[c, s], idxbuf)
    zero16 = jnp.zeros((16,), jnp.float32)

    def zfill(i, carry):
        for q in range(D // 16):
            zbuf[i, pl.ds(16 * q, 16)] = zero16
        return carry

    lax.fori_loop(0, ZR, zfill, 0)
    if with_deg:
        one16 = jnp.ones((16,), jnp.float32)

        def ofill(i, carry):
            for q in range(D // 16):
                onesbuf[i, pl.ds(16 * q, 16)] = one16
            return carry

        lax.fori_loop(0, CH, ofill, 0)
    for q in range(NPT // ZR):
        for a in accs:
            pltpu.sync_copy(zbuf, a.at[pl.ds(s * NPT + ZR * q, ZR)])
    plsc.subcore_barrier()
    for sup in range(NSUP):
        pltpu.sync_copy(acc5.at[c, s, pl.ds(SUP * sup, SUP)], vbuf)
        for m in range(SUP):
            idxrow = idxbuf.at[sup * SUP + m]
            pltpu.sync_copy(vbuf.at[m], accs[0].at[idxrow], add=True)
            if with_deg:
                pltpu.sync_copy(onesbuf, accs[1].at[idxrow], add=True)
    plsc.subcore_barrier()
    for a, o in zip(accs, outs):
        pltpu.sync_copy(a.at[pl.ds(s * NPT, NPT)], o.at[c, pl.ds(s * NPT, NPT)])


@functools.partial(
    pl.kernel,
    mesh=_MESH,
    out_type=jax.ShapeDtypeStruct((2, N, D), jnp.float32),
    scratch_types=[
        pltpu.VMEM((NCH, CH), jnp.int32),
        pltpu.VMEM((SUP, CH, D), jnp.float32),
        pltpu.VMEM((ZR, D), jnp.float32),
        pltpu.VMEM_SHARED((N, D), jnp.float32),
    ],
    compiler_params=_SC_PARAMS,
)
def _sc_scatter(acc5, dst4, out, idxbuf, vbuf, zbuf, acc_sh):
    _scatter_body(acc5, dst4, [out], idxbuf, vbuf, zbuf, None, [acc_sh], False)


@functools.partial(
    pl.kernel,
    mesh=_MESH,
    out_type=[
        jax.ShapeDtypeStruct((2, N, D), jnp.float32),
        jax.ShapeDtypeStruct((2, N, D), jnp.float32),
    ],
    scratch_types=[
        pltpu.VMEM((NCH, CH), jnp.int32),
        pltpu.VMEM((SUP, CH, D), jnp.float32),
        pltpu.VMEM((ZR, D), jnp.float32),
        pltpu.VMEM((CH, D), jnp.float32),
        pltpu.VMEM_SHARED((N, D), jnp.float32),
        pltpu.VMEM_SHARED((N, D), jnp.float32),
    ],
    compiler_params=_SC_PARAMS,
)
def _sc_scatter_deg(acc5, dst4, seg_out, deg_out, idxbuf, vbuf, zbuf, onesbuf,
                    acc_sh, deg_sh):
    _scatter_body(acc5, dst4, [seg_out, deg_out], idxbuf, vbuf, zbuf, onesbuf,
                  [acc_sh, deg_sh], True)


def _pre_body(x_ref, Wn_ref, bn_ref, Wh_ref, bh_ref, hin_ref):
    node = jax.lax.dot_general(x_ref[0], Wn_ref[0], (((1,), (0,)), ((), ())),
                               preferred_element_type=jnp.float32) + bn_ref[0]
    hid = jax.lax.dot_general(_leaky(node), Wh_ref[0], (((1,), (0,)), ((), ())),
                              preferred_element_type=jnp.float32) + bh_ref[0]
    hin_ref[0] = _leaky(hid)


def _tc_pre(xs, Wn, bn, Wh, bh):
    BN = 2000
    return pl.pallas_call(
        _pre_body,
        grid=(2, N // BN),
        in_specs=[
            pl.BlockSpec((1, BN, D_IN), lambda g, i: (g, i, 0)),
            pl.BlockSpec((1, D_IN, D_EMB), lambda g, i: (g, 0, 0)),
            pl.BlockSpec((1, 1, D_EMB), lambda g, i: (g, 0, 0)),
            pl.BlockSpec((1, D_EMB, D), lambda g, i: (g, 0, 0)),
            pl.BlockSpec((1, 1, D), lambda g, i: (g, 0, 0)),
        ],
        out_specs=pl.BlockSpec((1, BN, D), lambda g, i: (g, i, 0)),
        out_shape=jax.ShapeDtypeStruct((2, N, D), jnp.float32),
    )(xs, Wn, bn, Wh, bh)


def _edge_body(ea_ref, hs_ref, Wall_ref, We_ref, be_ref, out_ref):
    eT = jax.lax.dot_general(We_ref[0], ea_ref[0], (((0,), (1,)), ((), ())),
                             preferred_element_type=jnp.float32) + be_ref[0]
    PT = jax.lax.dot_general(Wall_ref[0], hs_ref[0], (((0,), (1,)), ((), ())),
                             preferred_element_type=jnp.float32)
    accT = PT[:D, :]
    for k in range(D_E):
        accT = accT + eT[k:k + 1, :] * PT[D * (k + 1):D * (k + 2), :]
    out_ref[0] = accT.T


def _tc_edge(eas, hs3, Wall_l, We, be):
    BE = 2000
    return pl.pallas_call(
        _edge_body,
        grid=(2, E // BE),
        in_specs=[
            pl.BlockSpec((1, BE, D_E), lambda g, i: (g, i, 0)),
            pl.BlockSpec((1, BE, D), lambda g, i: (g, i, 0)),
            pl.BlockSpec((1, D, 17 * D), lambda g, i: (g, 0, 0)),
            pl.BlockSpec((1, D_E, D_E), lambda g, i: (g, 0, 0)),
            pl.BlockSpec((1, D_E, 1), lambda g, i: (g, 0, 0)),
        ],
        out_specs=pl.BlockSpec((1, BE, D), lambda g, i: (g, i, 0)),
        out_shape=jax.ShapeDtypeStruct((2, E, D), jnp.float32),
    )(eas, hs3, Wall_l, We, be)


def _node_body_plain(seg_ref, deg_ref, hin_ref, Wr_ref, br_ref, Wo_ref, bo_ref,
                     hid_ref, hnext_ref):
    mean = seg_ref[0] / jnp.maximum(deg_ref[0], 1.0)
    conv = mean + jax.lax.dot_general(
        hin_ref[0], Wr_ref[0], (((1,), (0,)), ((), ())),
        preferred_element_type=jnp.float32) + br_ref[0]
    hid = jax.lax.dot_general(_leaky(conv), Wo_ref[0], (((1,), (0,)), ((), ())),
                              preferred_element_type=jnp.float32) + bo_ref[0]
    hid_ref[0] = hid
    hnext_ref[0] = _leaky(hid)


def _node_body_resid(seg_ref, deg_ref, hin_ref, Wr_ref, br_ref, Wo_ref, bo_ref,
                     res_ref, hid_ref, hnext_ref):
    mean = seg_ref[0] / jnp.maximum(deg_ref[0], 1.0)
    conv = mean + jax.lax.dot_general(
        hin_ref[0], Wr_ref[0], (((1,), (0,)), ((), ())),
        preferred_element_type=jnp.float32) + br_ref[0]
    hid = jax.lax.dot_general(_leaky(conv), Wo_ref[0], (((1,), (0,)), ((), ())),
                              preferred_element_type=jnp.float32) + bo_ref[0]
    hid = hid + res_ref[0]
    hid_ref[0] = hid
    hnext_ref[0] = _leaky(hid)


def _tc_node(seg, deg, hin, Wr, br, Wo, bo, resid=None):
    BN = 2000
    nmap = lambda g, i: (g, i, 0)
    wmap = lambda g, i: (g, 0, 0)
    in_specs = [
        pl.BlockSpec((1, BN, D), nmap),
        pl.BlockSpec((1, BN, D), nmap),
        pl.BlockSpec((1, BN, D), nmap),
        pl.BlockSpec((1, D, D), wmap),
        pl.BlockSpec((1, 1, D), wmap),
        pl.BlockSpec((1, D, D), wmap),
        pl.BlockSpec((1, 1, D), wmap),
    ]
    args = [seg, deg, hin, Wr, br, Wo, bo]
    body = _node_body_plain
    if resid is not None:
        in_specs.append(pl.BlockSpec((1, BN, D), nmap))
        args.append(resid)
        body = _node_body_resid
    return pl.pallas_call(
        body,
        grid=(2, N // BN),
        in_specs=in_specs,
        out_specs=[
            pl.BlockSpec((1, BN, D), nmap),
            pl.BlockSpec((1, BN, D), nmap),
        ],
        out_shape=[
            jax.ShapeDtypeStruct((2, N, D), jnp.float32),
            jax.ShapeDtypeStruct((2, N, D), jnp.float32),
        ],
    )(*args)


def _outer_body(lig_ref, rec_ref, ops_ref, out_ref):
    ops = jax.lax.dot_general(lig_ref[0], rec_ref[0], (((1,), (1,)), ((), ())),
                              preferred_element_type=jnp.float32)
    ops_ref[0] = ops
    out_ref[0, 0, 0] = jnp.sum(ops) / (NB * NB)


def _outer(lig_hid, rec_hid):
    lig3 = lig_hid.reshape(B, NB, D)
    rec3 = rec_hid.reshape(B, NB, D)
    ops, out = pl.pallas_call(
        _outer_body,
        grid=(B,),
        in_specs=[
            pl.BlockSpec((1, NB, D), lambda b: (b, 0, 0)),
            pl.BlockSpec((1, NB, D), lambda b: (b, 0, 0)),
        ],
        out_specs=[
            pl.BlockSpec((1, NB, NB), lambda b: (b, 0, 0)),
            pl.BlockSpec((1, 1, 1), lambda b: (b, 0, 0), memory_space=pltpu.SMEM),
        ],
        out_shape=[
            jax.ShapeDtypeStruct((B, NB, NB), jnp.float32),
            jax.ShapeDtypeStruct((B, 1, 1), jnp.float32),
        ],
    )(lig3, rec3)
    return out.reshape(B), ops


def kernel(lig_x, lig_edge_index, lig_edge_attr, rec_x, rec_edge_index, rec_edge_attr,
           lig_emb_Wn, lig_emb_bn, lig_emb_We, lig_emb_be, lig_hid_W, lig_hid_b,
           lig_Wedge, lig_bedge, lig_Wroot, lig_broot, lig_Wout, lig_bout,
           rec_emb_Wn, rec_emb_bn, rec_emb_We, rec_emb_be, rec_hid_W, rec_hid_b,
           rec_Wedge, rec_bedge, rec_Wroot, rec_broot, rec_Wout, rec_bout):
    f32 = jnp.float32
    xs = jnp.stack([lig_x, rec_x])
    eas = jnp.stack([lig_edge_attr, rec_edge_attr])
    ei = jnp.stack([lig_edge_index, rec_edge_index])
    src4 = ei[:, 0].reshape(2, NTEC, NCH, CH)
    dst4 = ei[:, 1].reshape(2, NTEC, NCH, CH)

    Wn = jnp.stack([lig_emb_Wn, rec_emb_Wn])
    bn = jnp.stack([lig_emb_bn, rec_emb_bn]).reshape(2, 1, D_EMB)
    We = jnp.stack([lig_emb_We, rec_emb_We])
    be = jnp.stack([lig_emb_be, rec_emb_be]).reshape(2, D_E, 1)
    Wh = jnp.stack([lig_hid_W, rec_hid_W])
    bh = jnp.stack([lig_hid_b, rec_hid_b]).reshape(2, 1, D)
    Wedge = jnp.stack([lig_Wedge, rec_Wedge])
    bedge = jnp.stack([lig_bedge, rec_bedge])
    T = jnp.concatenate([bedge[:, :, None], Wedge], axis=2)
    Wall = jnp.transpose(T, (0, 1, 3, 2, 4)).reshape(2, L, D, 17 * D)
    Wroot = jnp.stack([lig_Wroot, rec_Wroot])
    broot = jnp.stack([lig_broot, rec_broot]).reshape(2, L, 1, D)
    Wout = jnp.stack([lig_Wout, rec_Wout])
    bout = jnp.stack([lig_bout, rec_bout]).reshape(2, L, 1, D)

    hin = _tc_pre(xs, Wn, bn, Wh, bh)
    deg = None
    prev0 = None
    hid = None
    for l in range(L):
        table = hin.reshape(2 * N, D)
        hs = _sc_gather(table, src4)
        acc = _tc_edge(eas, hs.reshape(2, E, D), Wall[:, l], We, be)
        acc5 = acc.reshape(2, NTEC, NCH, CH, D)
        if l == 0:
            seg, deg = _sc_scatter_deg(acc5, dst4)
        else:
            seg = _sc_scatter(acc5, dst4)
        resid = prev0 if l == 2 else None
        hid, hin = _tc_node(seg, deg, hin, Wroot[:, l], broot[:, l],
                            Wout[:, l], bout[:, l], resid)
        if l == 0:
            prev0 = hid
    return _outer(hid[0], hid[1])
